# baseline (speedup 1.0000x reference)
import functools

import jax
import jax.numpy as jnp
from jax import lax
from jax.experimental import pallas as pl
from jax.experimental.pallas import tpu as pltpu
from jax.experimental.pallas import tpu_sc as plsc

_E = 16
_TOPK = 2
_HID = 1024
_FF = 2048
_T = 128
_NP = _T * _TOPK
_R = 32
_PADR = _NP + _R
_FBLK = 1024
_NF = _FF // _FBLK

_NC, _NS = 2, 16
_NW = _NC * _NS
_RPW = _NP // _NW


@functools.partial(
    pl.kernel,
    compiler_params=pltpu.CompilerParams(needs_layout_passes=False),
    out_type=[
        jax.ShapeDtypeStruct((_NP, _HID), jnp.float32),
        jax.ShapeDtypeStruct((_NP,), jnp.int32),
        jax.ShapeDtypeStruct((_NP,), jnp.float32),
        jax.ShapeDtypeStruct((_NP,), jnp.int32),
    ],
    mesh=plsc.VectorSubcoreMesh(core_axis_name="c", subcore_axis_name="s",
                                num_cores=_NC, num_subcores=_NS),
    scratch_types=[
        pltpu.VMEM((_NP,), jnp.int32),
        pltpu.VMEM((_NP,), jnp.float32),
        pltpu.VMEM((16,), jnp.int32),
        pltpu.VMEM((16,), jnp.int32),
        pltpu.VMEM((16,), jnp.int32),
        pltpu.VMEM((_NP,), jnp.int32),
        pltpu.VMEM((_NP,), jnp.float32),
        pltpu.VMEM((_NP,), jnp.int32),
        pltpu.VMEM((_RPW, _HID), jnp.float32),
        pltpu.SemaphoreType.DMA,
    ],
)
def _sc_route_gather(se_hbm, rw_hbm, x_hbm, xs_hbm, tok_hbm, w_hbm, es_hbm,
                     se_v, rw_v, cnt_v, off_v, run_v, tok_v, wv_v, es_v,
                     rows_v, sem):
    wid = lax.axis_index("s") * _NC + lax.axis_index("c")
    pltpu.sync_copy(se_hbm, se_v)
    pltpu.sync_copy(rw_hbm, rw_v)
    cnt_v[...] = jnp.zeros((16,), jnp.int32)
    run_v[...] = jnp.zeros((16,), jnp.int32)
    ones16 = jnp.ones((16,), jnp.int32)
    for c in range(16):
        e = se_v[pl.ds(c * 16, 16)]
        plsc.addupdate_scatter(cnt_v, [e], ones16)
    cnts = cnt_v[...]
    off_v[...] = plsc.cumsum(cnts) - cnts
    for c in range(16):
        e = se_v[pl.ds(c * 16, 16)]
        occ, _ = plsc.scan_count(e)
        base = plsc.load_gather(off_v, [e])
        run = plsc.load_gather(run_v, [e])
        rank = base + run + occ - 1
        tok = lax.div(lax.iota(jnp.int32, 16) + c * 16, _TOPK)
        plsc.store_scatter(tok_v, [rank], tok)
        plsc.store_scatter(wv_v, [rank], rw_v[pl.ds(c * 16, 16)])
        plsc.store_scatter(es_v, [rank], e)
        plsc.addupdate_scatter(run_v, [e], ones16)
    base_row = wid * _RPW
    pltpu.async_copy(x_hbm.at[tok_v.at[pl.ds(base_row, _RPW)]], rows_v,
                     sem).wait()
    pltpu.sync_copy(rows_v, xs_hbm.at[pl.ds(base_row, _RPW)])

    @pl.when(wid == 0)
    def _():
        pltpu.sync_copy(tok_v, tok_hbm)
        pltpu.sync_copy(wv_v, w_hbm)
        pltpu.sync_copy(es_v, es_hbm)


def _moe_body(se_ref, tok_ref, ws_ref, es_in_ref, xs_in_ref,
              gate_ref, up_ref, down_ref, out_ref,
              xs_ref, o_ref, p_ref, es_ref):
    e = pl.program_id(0)
    f = pl.program_id(1)

    @pl.when(jnp.logical_and(e == 0, f == 0))
    def _prologue():
        xs_ref[0:_NP, :] = xs_in_ref[...]
        xs_ref[_NP:_PADR, :] = jnp.zeros((_R, _HID), jnp.float32)
        o_ref[...] = jnp.zeros((_PADR, _HID), jnp.float32)
        tok_col = tok_ref[...].astype(jnp.float32)
        tcols = lax.broadcasted_iota(jnp.int32, (_NP, _T), 1).astype(jnp.float32)
        p_ref[...] = (jnp.broadcast_to(tok_col, (_NP, _T)) == tcols).astype(
            jnp.float32)
        es_col = es_in_ref[...].astype(jnp.float32)
        es_ref[0:_NP, :] = jnp.broadcast_to(es_col, (_NP, 128))
        es_ref[_NP:_PADR, :] = -jnp.ones((_R, 128), jnp.float32)

    se_all = se_ref[...]
    cnt = jnp.sum((se_all == e).astype(jnp.int32))
    start = jnp.sum((se_all < e).astype(jnp.int32))
    astart = (start // 8) * 8
    ntiles = ((start - astart) + cnt + _R - 1) // _R
    gate = gate_ref[...]
    up = up_ref[...]
    down = down_ref[...]
    e_f32 = e.astype(jnp.float32)

    def _tile(j, carry):
        base = pl.multiple_of(astart + j * _R, 8)
        tile = xs_ref[pl.ds(base, _R), :]
        g = jnp.dot(tile, gate, preferred_element_type=jnp.float32)
        u = jnp.dot(tile, up, preferred_element_type=jnp.float32)
        h = jax.nn.gelu(g, approximate=True) * u
        o = jnp.dot(h, down, preferred_element_type=jnp.float32)
        m = (es_ref[pl.ds(base, _R), 0:1] == e_f32).astype(jnp.float32)
        o_ref[pl.ds(base, _R), :] += o * m
        return carry

    lax.fori_loop(0, ntiles, _tile, 0)

    @pl.when(jnp.logical_and(e == _E - 1, f == _NF - 1))
    def _epilogue():
        o_scaled = o_ref[0:_NP, :] * ws_ref[...]
        out_ref[...] = lax.dot_general(p_ref[...], o_scaled,
                                       (((0,), (0,)), ((), ())),
                                       preferred_element_type=jnp.float32)


def _tc_moe(se, tok, ws, es, xs, gate_w, up_w, down_w):
    return pl.pallas_call(
        _moe_body,
        grid=(_E, _NF),
        in_specs=[
            pl.BlockSpec((1, _NP), lambda e, f: (0, 0)),
            pl.BlockSpec((_NP, 1), lambda e, f: (0, 0)),
            pl.BlockSpec((_NP, 1), lambda e, f: (0, 0)),
            pl.BlockSpec((_NP, 1), lambda e, f: (0, 0)),
            pl.BlockSpec((_NP, _HID), lambda e, f: (0, 0)),
            pl.BlockSpec((None, _HID, _FBLK), lambda e, f: (e, 0, f)),
            pl.BlockSpec((None, _HID, _FBLK), lambda e, f: (e, 0, f)),
            pl.BlockSpec((None, _FBLK, _HID), lambda e, f: (e, f, 0)),
        ],
        out_specs=pl.BlockSpec((_T, _HID), lambda e, f: (0, 0)),
        out_shape=jax.ShapeDtypeStruct((_T, _HID), jnp.float32),
        scratch_shapes=[
            pltpu.VMEM((_PADR, _HID), jnp.float32),
            pltpu.VMEM((_PADR, _HID), jnp.float32),
            pltpu.VMEM((_NP, _T), jnp.float32),
            pltpu.VMEM((_PADR, 128), jnp.float32),
        ],
        compiler_params=pltpu.CompilerParams(
            dimension_semantics=("arbitrary", "arbitrary"),
        ),
    )(se, tok, ws, es, xs, gate_w, up_w, down_w)


def kernel(x, selected_experts, routing_weights, gate_w, up_w, down_w):
    se_flat = selected_experts.reshape(_NP).astype(jnp.int32)
    rw_flat = routing_weights.reshape(_NP).astype(jnp.float32)
    xs, tok, ws, es = _sc_route_gather(se_flat, rw_flat, x)
    return _tc_moe(se_flat.reshape(1, _NP), tok.reshape(_NP, 1),
                   ws.reshape(_NP, 1), es.reshape(_NP, 1),
                   xs, gate_w, up_w, down_w)


# speedup vs baseline: 1.0099x; 1.0099x over previous
import functools

import jax
import jax.numpy as jnp
from jax import lax
from jax.experimental import pallas as pl
from jax.experimental.pallas import tpu as pltpu
from jax.experimental.pallas import tpu_sc as plsc

_E = 16
_TOPK = 2
_HID = 1024
_FF = 2048
_T = 128
_NP = _T * _TOPK
_R = 32
_PADR = _NP + _R
_FBLK = 1024
_NF = _FF // _FBLK

_NC, _NS = 2, 16
_NW = _NC * _NS
_RPW = _NP // _NW


@functools.partial(
    pl.kernel,
    compiler_params=pltpu.CompilerParams(needs_layout_passes=False),
    out_type=[
        jax.ShapeDtypeStruct((_NP,), jnp.int32),
        jax.ShapeDtypeStruct((_NP,), jnp.float32),
        jax.ShapeDtypeStruct((_NP,), jnp.int32),
    ],
    mesh=plsc.VectorSubcoreMesh(core_axis_name="c", subcore_axis_name="s",
                                num_cores=_NC, num_subcores=_NS),
    scratch_types=[
        pltpu.VMEM((_NP,), jnp.int32),
        pltpu.VMEM((_NP,), jnp.float32),
        pltpu.VMEM((16,), jnp.int32),
        pltpu.VMEM((16,), jnp.int32),
        pltpu.VMEM((16,), jnp.int32),
        pltpu.VMEM((_NP,), jnp.int32),
        pltpu.VMEM((_NP,), jnp.float32),
        pltpu.VMEM((_NP,), jnp.int32),
    ],
)
def _sc_route_gather(se_hbm, rw_hbm, tok_hbm, w_hbm, es_hbm,
                     se_v, rw_v, cnt_v, off_v, run_v, tok_v, wv_v, es_v):
    wid = lax.axis_index("s") * _NC + lax.axis_index("c")
    pltpu.sync_copy(se_hbm, se_v)
    pltpu.sync_copy(rw_hbm, rw_v)
    cnt_v[...] = jnp.zeros((16,), jnp.int32)
    run_v[...] = jnp.zeros((16,), jnp.int32)
    ones16 = jnp.ones((16,), jnp.int32)
    for c in range(16):
        e = se_v[pl.ds(c * 16, 16)]
        plsc.addupdate_scatter(cnt_v, [e], ones16)
    cnts = cnt_v[...]
    off_v[...] = plsc.cumsum(cnts) - cnts
    for c in range(16):
        e = se_v[pl.ds(c * 16, 16)]
        occ, _ = plsc.scan_count(e)
        base = plsc.load_gather(off_v, [e])
        run = plsc.load_gather(run_v, [e])
        rank = base + run + occ - 1
        tok = lax.div(lax.iota(jnp.int32, 16) + c * 16, _TOPK)
        plsc.store_scatter(tok_v, [rank], tok)
        plsc.store_scatter(wv_v, [rank], rw_v[pl.ds(c * 16, 16)])
        plsc.store_scatter(es_v, [rank], e)
        plsc.addupdate_scatter(run_v, [e], ones16)
    @pl.when(wid == 0)
    def _():
        pltpu.sync_copy(tok_v, tok_hbm)
        pltpu.sync_copy(wv_v, w_hbm)
        pltpu.sync_copy(es_v, es_hbm)


def _moe_body(se_ref, tok_ref, ws_ref, es_in_ref, x_ref,
              gate_ref, up_ref, down_ref, out_ref,
              xs_ref, o_ref, p_ref, es_ref):
    e = pl.program_id(0)
    f = pl.program_id(1)

    @pl.when(jnp.logical_and(e == 0, f == 0))
    def _prologue():
        tok_col = tok_ref[...].astype(jnp.float32)
        tcols = lax.broadcasted_iota(jnp.int32, (_NP, _T), 1).astype(jnp.float32)
        p_ref[...] = (jnp.broadcast_to(tok_col, (_NP, _T)) == tcols).astype(
            jnp.float32)
        xs_ref[0:_NP, :] = jnp.dot(p_ref[...], x_ref[...],
                                   preferred_element_type=jnp.float32)
        xs_ref[_NP:_PADR, :] = jnp.zeros((_R, _HID), jnp.float32)
        o_ref[...] = jnp.zeros((_PADR, _HID), jnp.float32)
        es_col = es_in_ref[...].astype(jnp.float32)
        es_ref[0:_NP, :] = jnp.broadcast_to(es_col, (_NP, 128))
        es_ref[_NP:_PADR, :] = -jnp.ones((_R, 128), jnp.float32)

    se_all = se_ref[...]
    cnt = jnp.sum((se_all == e).astype(jnp.int32))
    start = jnp.sum((se_all < e).astype(jnp.int32))
    astart = (start // 8) * 8
    ntiles = ((start - astart) + cnt + _R - 1) // _R
    gate = gate_ref[...]
    up = up_ref[...]
    down = down_ref[...]
    e_f32 = e.astype(jnp.float32)

    def _tile(j, carry):
        base = pl.multiple_of(astart + j * _R, 8)
        tile = xs_ref[pl.ds(base, _R), :]
        g = jnp.dot(tile, gate, preferred_element_type=jnp.float32)
        u = jnp.dot(tile, up, preferred_element_type=jnp.float32)
        h = jax.nn.gelu(g, approximate=True) * u
        o = jnp.dot(h, down, preferred_element_type=jnp.float32)
        m = (es_ref[pl.ds(base, _R), 0:1] == e_f32).astype(jnp.float32)
        o_ref[pl.ds(base, _R), :] += o * m
        return carry

    lax.fori_loop(0, ntiles, _tile, 0)

    @pl.when(jnp.logical_and(e == _E - 1, f == _NF - 1))
    def _epilogue():
        o_scaled = o_ref[0:_NP, :] * ws_ref[...]
        out_ref[...] = lax.dot_general(p_ref[...], o_scaled,
                                       (((0,), (0,)), ((), ())),
                                       preferred_element_type=jnp.float32)


def _tc_moe(se, tok, ws, es, x, gate_w, up_w, down_w):
    return pl.pallas_call(
        _moe_body,
        grid=(_E, _NF),
        in_specs=[
            pl.BlockSpec((1, _NP), lambda e, f: (0, 0)),
            pl.BlockSpec((_NP, 1), lambda e, f: (0, 0)),
            pl.BlockSpec((_NP, 1), lambda e, f: (0, 0)),
            pl.BlockSpec((_NP, 1), lambda e, f: (0, 0)),
            pl.BlockSpec((_T, _HID), lambda e, f: (0, 0)),
            pl.BlockSpec((None, _HID, _FBLK), lambda e, f: (e, 0, f)),
            pl.BlockSpec((None, _HID, _FBLK), lambda e, f: (e, 0, f)),
            pl.BlockSpec((None, _FBLK, _HID), lambda e, f: (e, f, 0)),
        ],
        out_specs=pl.BlockSpec((_T, _HID), lambda e, f: (0, 0)),
        out_shape=jax.ShapeDtypeStruct((_T, _HID), jnp.float32),
        scratch_shapes=[
            pltpu.VMEM((_PADR, _HID), jnp.float32),
            pltpu.VMEM((_PADR, _HID), jnp.float32),
            pltpu.VMEM((_NP, _T), jnp.float32),
            pltpu.VMEM((_PADR, 128), jnp.float32),
        ],
        compiler_params=pltpu.CompilerParams(
            dimension_semantics=("arbitrary", "arbitrary"),
        ),
    )(se, tok, ws, es, x, gate_w, up_w, down_w)


def kernel(x, selected_experts, routing_weights, gate_w, up_w, down_w):
    se_flat = selected_experts.reshape(_NP).astype(jnp.int32)
    rw_flat = routing_weights.reshape(_NP).astype(jnp.float32)
    tok, ws, es = _sc_route_gather(se_flat, rw_flat)
    return _tc_moe(se_flat.reshape(1, _NP), tok.reshape(_NP, 1),
                   ws.reshape(_NP, 1), es.reshape(_NP, 1),
                   x, gate_w, up_w, down_w)
